# Initial kernel scaffold; baseline (speedup 1.0000x reference)
import jax
import jax.numpy as jnp
from jax.experimental import pallas as pl


def kernel(a, t, v, mask, Wa_w, Wa_b, Wt_w, Wt_b, Wv_w, Wv_b, wa_w, wa_b, wt_w, wt_b, wv_w, wv_b, da_w, da_b, dt_w, dt_b, dv_w, dv_b):
    raise NotImplementedError("write your pallas kernel here")


# speedup vs baseline: 4.1560x; 4.1560x over previous
import jax
import jax.numpy as jnp
from jax import lax
from jax.experimental import pallas as pl
from jax.experimental.pallas import tpu as pltpu

N = 1024
K = 4
BIG = 1e30


def _dotT(x, w):
    return lax.dot_general(x, w, (((1,), (1,)), ((), ())),
                           preferred_element_type=jnp.float32)


def _encode_kernel(a_ref, t_ref, v_ref, wa_ref, ba_ref, wt_ref, bt_ref,
                   wv_ref, bv_ref, ea_ref, et_ref, ev_ref):
    ea_ref[...] = _dotT(a_ref[...], wa_ref[...]) + ba_ref[...]
    et_ref[...] = _dotT(t_ref[...], wt_ref[...]) + bt_ref[...]
    ev_ref[...] = _dotT(v_ref[...], wv_ref[...]) + bv_ref[...]


def _affinity_kernel(x_ref, mrow_ref, mcol_ref, p_ref):
    x = x_ref[...]
    sq = jnp.sum(x * x, axis=1, keepdims=True)
    g = _dotT(x, x)
    d2 = sq + sq.T - 2.0 * g
    d = jnp.sqrt(jnp.maximum(d2, 0.0) + 1e-12)
    iota = lax.broadcasted_iota(jnp.int32, (N, N), 1)
    rowi = lax.broadcasted_iota(jnp.int32, (N, N), 0)
    eye = iota == rowi
    a_mat = jnp.zeros((N, N), jnp.float32)
    dwork = d
    for _ in range(K):
        m = jnp.min(dwork, axis=1, keepdims=True)
        ismin = dwork == m
        jstar = jnp.min(jnp.where(ismin, iota, N), axis=1, keepdims=True)
        sel = iota == jstar
        a_mat = a_mat + jnp.where(sel, 1.0 / (1.0 + dwork), 0.0)
        dwork = jnp.where(sel, BIG, dwork)
    a_mat = a_mat * mrow_ref[...] * mcol_ref[...]
    a_mat = jnp.maximum(a_mat, a_mat.T)
    s_mat = jnp.where(eye, 2.0, a_mat)
    rs_col = jnp.sum(s_mat, axis=1, keepdims=True)
    rs_row = jnp.sum(s_mat, axis=0, keepdims=True)
    dc = lax.rsqrt(rs_col + 1e-12)
    dr = lax.rsqrt(rs_row + 1e-12)
    p_ref[...] = dc * s_mat * dr


def _mix_kernel(x_ref, p1_ref, p2_ref, o_ref):
    x = x_ref[...]
    ps = lax.dot_general(p1_ref[...] + p2_ref[...], x, (((1,), (0,)), ((), ())),
                         preferred_element_type=jnp.float32)
    o_ref[...] = 0.5 * x + 0.25 * ps


def _head_kernel(h_ref, w_ref, wb_ref, d_ref, db_ref, o_ref):
    z = _dotT(h_ref[...], w_ref[...]) + wb_ref[...]
    o_ref[...] = _dotT(z, d_ref[...]) + db_ref[...]


def kernel(a, t, v, mask, Wa_w, Wa_b, Wt_w, Wt_b, Wv_w, Wv_b,
           wa_w, wa_b, wt_w, wt_b, wv_w, wv_b,
           da_w, da_b, dt_w, dt_b, dv_w, dv_b):
    f32 = jnp.float32
    mrow = mask.reshape(1, N)
    mcol = mask.reshape(N, 1)

    ea, et, ev = pl.pallas_call(
        _encode_kernel,
        out_shape=[jax.ShapeDtypeStruct((N, 2048), f32)] * 3,
    )(a, t, v, Wa_w, Wa_b.reshape(1, -1), Wt_w, Wt_b.reshape(1, -1),
      Wv_w, Wv_b.reshape(1, -1))

    aff = pl.pallas_call(
        _affinity_kernel,
        out_shape=jax.ShapeDtypeStruct((N, N), f32),
    )
    pa = aff(ea, mrow, mcol)
    pt = aff(et, mrow, mcol)
    pv = aff(ev, mrow, mcol)

    mix = pl.pallas_call(
        _mix_kernel,
        out_shape=jax.ShapeDtypeStruct((N, 2048), f32),
    )
    ha = mix(ea, pt, pv)
    ht = mix(et, pv, pa)
    hv = mix(ev, pa, pt)

    def head(h, w, wb, d, db):
        return pl.pallas_call(
            _head_kernel,
            out_shape=jax.ShapeDtypeStruct((N, d.shape[0]), f32),
        )(h, w, wb.reshape(1, -1), d, db.reshape(1, -1))

    ra = head(ha, wa_w, wa_b, da_w, da_b)
    rt = head(ht, wt_w, wt_b, dt_w, dt_b)
    rv = head(hv, wv_w, wv_b, dv_w, dv_b)
    return jnp.concatenate([ra, rt, rv], axis=1)


# speedup vs baseline: 4.8772x; 1.1735x over previous
import jax
import jax.numpy as jnp
from jax import lax
from jax.experimental import pallas as pl
from jax.experimental.pallas import tpu as pltpu

N = 1024
ED = 2048
K = 4
BIG = 1e30


def _dotT(x, w):
    return lax.dot_general(x, w, (((1,), (1,)), ((), ())),
                           preferred_element_type=jnp.float32)


def _encode_kernel(a_ref, t_ref, v_ref, wa_ref, ba_ref, wt_ref, bt_ref,
                   wv_ref, bv_ref, ea_ref, et_ref, ev_ref):
    ea_ref[...] = _dotT(a_ref[...], wa_ref[...]) + ba_ref[...]
    et_ref[...] = _dotT(t_ref[...], wt_ref[...]) + bt_ref[...]
    ev_ref[...] = _dotT(v_ref[...], wv_ref[...]) + bv_ref[...]


def _affinity_kernel(x_ref, mrow_ref, mcol_ref, p_ref):
    x = x_ref[...]
    sq = jnp.sum(x * x, axis=1, keepdims=True)
    g = _dotT(x, x)
    d2 = sq + sq.T - 2.0 * g
    iota = lax.broadcasted_iota(jnp.int32, (N, N), 1)
    rowi = lax.broadcasted_iota(jnp.int32, (N, N), 0)
    eye = iota == rowi
    jstars = []
    sims = []
    dwork = d2
    for _ in range(K):
        m = jnp.min(dwork, axis=1, keepdims=True)
        jstar = jnp.min(jnp.where(dwork == m, iota, N), axis=1, keepdims=True)
        jstars.append(jstar)
        sims.append(1.0 / (1.0 + jnp.sqrt(jnp.maximum(m, 0.0) + 1e-12)))
        dwork = jnp.where(iota == jstar, BIG, dwork)
    a_mat = jnp.zeros((N, N), jnp.float32)
    for jstar, sim in zip(jstars, sims):
        a_mat = a_mat + jnp.where(iota == jstar, sim, 0.0)
    a_mat = a_mat * mrow_ref[...] * mcol_ref[...]
    a_mat = jnp.maximum(a_mat, a_mat.T)
    s_mat = jnp.where(eye, 2.0, a_mat)
    rs_col = jnp.sum(s_mat, axis=1, keepdims=True)
    dc = lax.rsqrt(rs_col + 1e-12)
    p_ref[...] = (dc * s_mat * dc.T).astype(jnp.bfloat16)


def _fold_kernel(w_ref, wb_ref, d_ref, db_ref, m_ref, b2_ref):
    wb16 = w_ref[...].astype(jnp.bfloat16)
    db16 = d_ref[...].astype(jnp.bfloat16)
    m_ref[...] = lax.dot_general(
        wb16, db16, (((0,), (1,)), ((), ())),
        preferred_element_type=jnp.float32).astype(jnp.bfloat16)
    b2_ref[...] = _dotT(wb_ref[...], d_ref[...]) + db_ref[...]


def _headmix_kernel(e_ref, m_ref, b2_ref, p1_ref, p2_ref, o_ref):
    y = lax.dot_general(e_ref[...].astype(jnp.bfloat16), m_ref[...],
                        (((1,), (0,)), ((), ())),
                        preferred_element_type=jnp.float32)
    mixed = lax.dot_general(p1_ref[...] + p2_ref[...], y.astype(jnp.bfloat16),
                            (((1,), (0,)), ((), ())),
                            preferred_element_type=jnp.float32)
    o_ref[...] = 0.5 * y + 0.25 * mixed + b2_ref[...]


def kernel(a, t, v, mask, Wa_w, Wa_b, Wt_w, Wt_b, Wv_w, Wv_b,
           wa_w, wa_b, wt_w, wt_b, wv_w, wv_b,
           da_w, da_b, dt_w, dt_b, dv_w, dv_b):
    f32 = jnp.float32
    bf16 = jnp.bfloat16
    mrow = mask.reshape(1, N)
    mcol = mask.reshape(N, 1)

    ea, et, ev = pl.pallas_call(
        _encode_kernel,
        out_shape=[jax.ShapeDtypeStruct((N, ED), f32)] * 3,
    )(a, t, v, Wa_w, Wa_b.reshape(1, -1), Wt_w, Wt_b.reshape(1, -1),
      Wv_w, Wv_b.reshape(1, -1))

    aff = pl.pallas_call(
        _affinity_kernel,
        out_shape=jax.ShapeDtypeStruct((N, N), bf16),
    )
    pa = aff(ea, mrow, mcol)
    pt = aff(et, mrow, mcol)
    pv = aff(ev, mrow, mcol)

    def fold(w, wb, d, db):
        dout = d.shape[0]
        return pl.pallas_call(
            _fold_kernel,
            out_shape=[jax.ShapeDtypeStruct((ED, dout), bf16),
                       jax.ShapeDtypeStruct((1, dout), f32)],
        )(w, wb.reshape(1, -1), d, db.reshape(1, -1))

    ma, b2a = fold(wa_w, wa_b, da_w, da_b)
    mt, b2t = fold(wt_w, wt_b, dt_w, dt_b)
    mv, b2v = fold(wv_w, wv_b, dv_w, dv_b)

    def headmix(eb, m, b2, p1, p2):
        return pl.pallas_call(
            _headmix_kernel,
            out_shape=jax.ShapeDtypeStruct((N, m.shape[1]), f32),
        )(eb, m, b2, p1, p2)

    ra = headmix(ea, ma, b2a, pt, pv)
    rt = headmix(et, mt, b2t, pv, pa)
    rv = headmix(ev, mv, b2v, pa, pt)
    return jnp.concatenate([ra, rt, rv], axis=1)
